# baseline (speedup 1.0000x reference)
import jax
import jax.numpy as jnp
import numpy as np
from jax.experimental import pallas as pl
from jax.experimental.pallas import tpu as pltpu

B, S, D, H = 4, 4096, 1024, 16
DH = D // H
BUCKET = 64
NH = 4
NCH = S // BUCKET
NB = NCH
DFF = 4096


def _ffn_body(a_ref, wo_ref, w1_ref, b1_ref, w2_ref, b2_ref, o_ref):
    a = a_ref[...]
    x = jnp.dot(a, wo_ref[...], preferred_element_type=jnp.float32)
    h = jnp.maximum(
        jnp.dot(x, w1_ref[...], preferred_element_type=jnp.float32) + b1_ref[...],
        0.0,
    )
    o_ref[...] = jnp.dot(h, w2_ref[...], preferred_element_type=jnp.float32) + b2_ref[...]


def _ffn(attn, Wo, W1, b1, W2, b2):
    N = B * S
    TILE = 512
    a2 = attn.reshape(N, D)
    out = pl.pallas_call(
        _ffn_body,
        grid=(N // TILE,),
        in_specs=[
            pl.BlockSpec((TILE, D), lambda i: (i, 0)),
            pl.BlockSpec((D, D), lambda i: (0, 0)),
            pl.BlockSpec((D, DFF), lambda i: (0, 0)),
            pl.BlockSpec((1, DFF), lambda i: (0, 0)),
            pl.BlockSpec((DFF, D), lambda i: (0, 0)),
            pl.BlockSpec((1, D), lambda i: (0, 0)),
        ],
        out_specs=pl.BlockSpec((TILE, D), lambda i: (i, 0)),
        out_shape=jax.ShapeDtypeStruct((N, D), jnp.float32),
    )(a2, Wo, W1, b1.reshape(1, DFF), W2, b2.reshape(1, D))
    return out.reshape(B, S, D)


def kernel(src, Wqk, Wv, Wo, rot, W1, b1, W2, b2):
    qk = (src @ Wqk).reshape(B, S, H, DH).transpose(0, 2, 1, 3)
    v = (src @ Wv).reshape(B, S, H, DH).transpose(0, 2, 1, 3)
    qn = qk / (jnp.linalg.norm(qk, axis=-1, keepdims=True) + 1e-6)
    rotated = jnp.einsum('bhsd,dnr->bhsnr', qn, rot)
    buckets = jnp.argmax(jnp.concatenate([rotated, -rotated], axis=-1), axis=-1)
    pos = jnp.arange(S)
    scale = 1.0 / np.sqrt(DH)
    eye = jnp.eye(BUCKET, dtype=bool)
    self_mask = jnp.concatenate([eye, jnp.zeros((BUCKET, BUCKET), bool)], axis=-1)
    outs, lses = [], []
    for r in range(NH):
        b_r = buckets[..., r]
        skey = b_r * S + pos
        order = jnp.argsort(skey, axis=-1)
        inv = jnp.argsort(order, axis=-1)
        qk_s = jnp.take_along_axis(qk, order[..., None], axis=2)
        v_s = jnp.take_along_axis(v, order[..., None], axis=2)
        qc = qk_s.reshape(B, H, NCH, BUCKET, DH)
        kc = qc / (jnp.linalg.norm(qc, axis=-1, keepdims=True) + 1e-6)
        vc = v_s.reshape(B, H, NCH, BUCKET, DH)
        kcat = jnp.concatenate([kc, jnp.roll(kc, 1, axis=2)], axis=3)
        vcat = jnp.concatenate([vc, jnp.roll(vc, 1, axis=2)], axis=3)
        dots = jnp.einsum('bhcqd,bhckd->bhcqk', qc, kcat) * scale
        dots = jnp.where(self_mask, -1e5, dots)
        lse = jax.nn.logsumexp(dots, axis=-1)
        probs = jnp.exp(dots - lse[..., None])
        oc = jnp.einsum('bhcqk,bhckd->bhcqd', probs, vcat)
        o = oc.reshape(B, H, S, DH)
        l = lse.reshape(B, H, S)
        o = jnp.take_along_axis(o, inv[..., None], axis=2)
        l = jnp.take_along_axis(l, inv, axis=2)
        outs.append(o)
        lses.append(l)
    o_stack = jnp.stack(outs, 0)
    l_stack = jnp.stack(lses, 0)
    w = jax.nn.softmax(l_stack, axis=0)[..., None]
    attn = jnp.sum(o_stack * w, axis=0)
    attn = attn.transpose(0, 2, 1, 3).reshape(B, S, D)
    return _ffn(attn, Wo, W1, b1, W2, b2)


# speedup vs baseline: 3.2103x; 3.2103x over previous
import functools

import jax
import jax.numpy as jnp
import numpy as np
from jax import lax
from jax.experimental import pallas as pl
from jax.experimental.pallas import tpu as pltpu
from jax.experimental.pallas import tpu_sc as plsc

B, S, D, H = 4, 4096, 1024, 16
DH = D // H
BUCKET = 64
NH = 4
NCH = S // BUCKET
NB = NCH
DFF = 4096
BH = B * H
T = NH * BH
SB = 512
QKV = 2 * DH
OLSE = 128


def _dest_body(bucket_ref, dest_ref):
    bh = pl.program_id(0)
    r = pl.program_id(1)
    bcol = bucket_ref[0, 0]
    lane = lax.broadcasted_iota(jnp.int32, (SB, BUCKET), 1)
    tri_s = (lax.broadcasted_iota(jnp.int32, (SB, SB), 0)
             > lax.broadcasted_iota(jnp.int32, (SB, SB), 1)).astype(jnp.bfloat16)
    tri_b = (lax.broadcasted_iota(jnp.int32, (BUCKET, BUCKET), 0)
             < lax.broadcasted_iota(jnp.int32, (BUCKET, BUCKET), 1)).astype(jnp.float32)
    tot = jnp.zeros((1, BUCKET), jnp.float32)
    ohs = []
    for j in range(S // SB):
        oh = (bcol[j * SB:(j + 1) * SB] == lane).astype(jnp.float32)
        ohs.append(oh)
        tot = tot + jnp.sum(oh, axis=0, keepdims=True)
    offs = jnp.dot(tot, tri_b, preferred_element_type=jnp.float32)
    tbase = ((r * BH + bh) * S).astype(jnp.float32)
    carry = jnp.zeros((1, BUCKET), jnp.float32)
    for j in range(S // SB):
        oh = ohs[j]
        cums = jnp.dot(tri_s, oh.astype(jnp.bfloat16),
                       preferred_element_type=jnp.float32)
        destc = jnp.sum(oh * (offs + carry + cums), axis=1, keepdims=True)
        dest_ref[0, 0, j * SB:(j + 1) * SB] = (destc + tbase).astype(jnp.int32)
        carry = carry + jnp.sum(oh, axis=0, keepdims=True)


def _dest(bucket4):
    return pl.pallas_call(
        _dest_body,
        grid=(BH, NH),
        in_specs=[pl.BlockSpec((1, 1, S, 1), lambda bh, r: (r, bh, 0, 0))],
        out_specs=pl.BlockSpec((1, 1, S, 1), lambda bh, r: (r, bh, 0, 0)),
        out_shape=jax.ShapeDtypeStruct((NH, BH, S, 1), jnp.int32),
    )(bucket4)


_NW = 32
_IPW = T // _NW
_CH = 512
_NCHK = S // _CH
_KJ = _CH // 128

@functools.cache
def _sc_kernels():
    mesh = plsc.VectorSubcoreMesh(core_axis_name="c", subcore_axis_name="s")

    @functools.partial(
        pl.kernel, mesh=mesh,
        out_type=jax.ShapeDtypeStruct((T * S, QKV), jnp.float32),
        scratch_types=[
            pltpu.VMEM((_KJ, 128), jnp.int32),
            pltpu.VMEM((_CH, QKV), jnp.float32),
            pltpu.SemaphoreType.DMA,
        ],
    )
    def sc_scatter(qkv_hbm, dest_hbm, sorted_hbm, idx_v, rows_v, sem):
        wid = lax.axis_index("s") * 2 + lax.axis_index("c")

        def inst_body(k, carry):
            t = wid * _IPW + k
            bh = lax.rem(t, BH)

            def chunk_body(c, carry2):
                pltpu.sync_copy(dest_hbm.at[t, pl.ds(c * _KJ, _KJ)], idx_v)
                pltpu.sync_copy(qkv_hbm.at[pl.ds(bh * S + c * _CH, _CH)], rows_v)
                cps = [
                    pltpu.async_copy(rows_v.at[pl.ds(j * 128, 128)],
                                     sorted_hbm.at[idx_v.at[j]], sem)
                    for j in range(_KJ)
                ]
                for cp in cps:
                    cp.wait()
                return carry2

            return lax.fori_loop(0, _NCHK, chunk_body, carry)

        lax.fori_loop(0, _IPW, inst_body, 0)

    @functools.partial(
        pl.kernel, mesh=mesh,
        out_type=jax.ShapeDtypeStruct((T * S, OLSE), jnp.float32),
        scratch_types=[
            pltpu.VMEM((_KJ, 128), jnp.int32),
            pltpu.VMEM((_CH, OLSE), jnp.float32),
            pltpu.SemaphoreType.DMA,
        ],
    )
    def sc_gather(olse_hbm, dest_hbm, out_hbm, idx_v, rows_v, sem):
        wid = lax.axis_index("s") * 2 + lax.axis_index("c")

        def inst_body(k, carry):
            t = wid * _IPW + k

            def chunk_body(c, carry2):
                pltpu.sync_copy(dest_hbm.at[t, pl.ds(c * _KJ, _KJ)], idx_v)
                cps = [
                    pltpu.async_copy(olse_hbm.at[idx_v.at[j]],
                                     rows_v.at[pl.ds(j * 128, 128)], sem)
                    for j in range(_KJ)
                ]
                for cp in cps:
                    cp.wait()
                pltpu.sync_copy(rows_v, out_hbm.at[pl.ds(t * S + c * _CH, _CH)])
                return carry2

            return lax.fori_loop(0, _NCHK, chunk_body, carry)

        lax.fori_loop(0, _IPW, inst_body, 0)

    return sc_scatter, sc_gather


def _ffn_body(a_ref, wo_ref, w1_ref, b1_ref, w2_ref, b2_ref, o_ref):
    a = a_ref[...]
    x = jnp.dot(a, wo_ref[...], preferred_element_type=jnp.float32)
    h = jnp.maximum(
        jnp.dot(x, w1_ref[...], preferred_element_type=jnp.float32) + b1_ref[...],
        0.0,
    )
    o_ref[...] = jnp.dot(h, w2_ref[...], preferred_element_type=jnp.float32) + b2_ref[...]


def _ffn(attn, Wo, W1, b1, W2, b2):
    N = B * S
    TILE = 512
    out = pl.pallas_call(
        _ffn_body,
        grid=(N // TILE,),
        in_specs=[
            pl.BlockSpec((TILE, D), lambda i: (i, 0)),
            pl.BlockSpec((D, D), lambda i: (0, 0)),
            pl.BlockSpec((D, DFF), lambda i: (0, 0)),
            pl.BlockSpec((1, DFF), lambda i: (0, 0)),
            pl.BlockSpec((DFF, D), lambda i: (0, 0)),
            pl.BlockSpec((1, D), lambda i: (0, 0)),
        ],
        out_specs=pl.BlockSpec((TILE, D), lambda i: (i, 0)),
        out_shape=jax.ShapeDtypeStruct((N, D), jnp.float32),
    )(attn.reshape(N, D), Wo, W1, b1.reshape(1, DFF), W2, b2.reshape(1, D))
    return out


def kernel(src, Wqk, Wv, Wo, rot, W1, b1, W2, b2):
    qk = (src @ Wqk).reshape(B, S, H, DH).transpose(0, 2, 1, 3)
    v = (src @ Wv).reshape(B, S, H, DH).transpose(0, 2, 1, 3)
    qn = qk / (jnp.linalg.norm(qk, axis=-1, keepdims=True) + 1e-6)
    rotated = jnp.einsum('bhsd,dnr->bhsnr', qn, rot)
    buckets = jnp.argmax(
        jnp.concatenate([rotated, -rotated], axis=-1), axis=-1
    ).astype(jnp.int32)

    bucket4 = buckets.transpose(3, 0, 1, 2).reshape(NH, BH, S, 1)
    dest_g = _dest(bucket4)
    dest_sc = dest_g.reshape(T, S // 128, 128)

    sc_scatter, sc_gather = _sc_kernels()
    qkv = jnp.concatenate([qk, v], axis=-1).reshape(BH * S, QKV)
    srt = sc_scatter(qkv, dest_sc).reshape(NH, B, H, NCH, BUCKET, QKV)

    qc = srt[..., :DH]
    vc = srt[..., DH:]
    kc = qc / (jnp.linalg.norm(qc, axis=-1, keepdims=True) + 1e-6)
    kcat = jnp.concatenate([kc, jnp.roll(kc, 1, axis=3)], axis=4)
    vcat = jnp.concatenate([vc, jnp.roll(vc, 1, axis=3)], axis=4)
    scale = 1.0 / np.sqrt(DH)
    dots = jnp.einsum('nbhcqd,nbhckd->nbhcqk', qc, kcat) * scale
    eye = jnp.eye(BUCKET, dtype=bool)
    self_mask = jnp.concatenate([eye, jnp.zeros((BUCKET, BUCKET), bool)], axis=-1)
    dots = jnp.where(self_mask, -1e5, dots)
    lse = jax.nn.logsumexp(dots, axis=-1)
    probs = jnp.exp(dots - lse[..., None])
    oc = jnp.einsum('nbhcqk,nbhckd->nbhcqd', probs, vcat)

    olse = jnp.concatenate(
        [oc.reshape(T * S, DH), lse.reshape(T * S, 1),
         jnp.zeros((T * S, OLSE - DH - 1), jnp.float32)], axis=-1)
    uns = sc_gather(olse, dest_sc).reshape(NH, B, H, S, OLSE)

    o_u = uns[..., :DH]
    l_u = uns[..., DH]
    w = jax.nn.softmax(l_u, axis=0)[..., None]
    attn = jnp.sum(o_u * w, axis=0)
    attn = attn.transpose(0, 2, 1, 3).reshape(B * S, D)
    return _ffn(attn, Wo, W1, b1, W2, b2).reshape(B, S, D)


# speedup vs baseline: 3.2720x; 1.0192x over previous
import functools

import jax
import jax.numpy as jnp
import numpy as np
from jax import lax
from jax.experimental import pallas as pl
from jax.experimental.pallas import tpu as pltpu
from jax.experimental.pallas import tpu_sc as plsc

B, S, D, H = 4, 4096, 1024, 16
DH = D // H
BUCKET = 64
NH = 4
NCH = S // BUCKET
NB = NCH
DFF = 4096
BH = B * H
T = NH * BH
SB = 512
QKV = 2 * DH
OLSE = 128


def _dest_body(bucket_ref, dest_ref):
    bh = pl.program_id(0)
    r = pl.program_id(1)
    bcol = bucket_ref[0, 0]
    lane = lax.broadcasted_iota(jnp.int32, (SB, BUCKET), 1)
    tri_s = (lax.broadcasted_iota(jnp.int32, (SB, SB), 0)
             > lax.broadcasted_iota(jnp.int32, (SB, SB), 1)).astype(jnp.bfloat16)
    tri_b = (lax.broadcasted_iota(jnp.int32, (BUCKET, BUCKET), 0)
             < lax.broadcasted_iota(jnp.int32, (BUCKET, BUCKET), 1)).astype(jnp.float32)
    tot = jnp.zeros((1, BUCKET), jnp.float32)
    ohs = []
    for j in range(S // SB):
        oh = (bcol[j * SB:(j + 1) * SB] == lane).astype(jnp.float32)
        ohs.append(oh)
        tot = tot + jnp.sum(oh, axis=0, keepdims=True)
    offs = jnp.dot(tot, tri_b, preferred_element_type=jnp.float32)
    tbase = ((r * BH + bh) * S).astype(jnp.float32)
    carry = jnp.zeros((1, BUCKET), jnp.float32)
    for j in range(S // SB):
        oh = ohs[j]
        cums = jnp.dot(tri_s, oh.astype(jnp.bfloat16),
                       preferred_element_type=jnp.float32)
        destc = jnp.sum(oh * (offs + carry + cums), axis=1, keepdims=True)
        dest_ref[0, 0, j * SB:(j + 1) * SB] = (destc + tbase).astype(jnp.int32)
        carry = carry + jnp.sum(oh, axis=0, keepdims=True)


def _dest(bucket4):
    return pl.pallas_call(
        _dest_body,
        grid=(BH, NH),
        in_specs=[pl.BlockSpec((1, 1, S, 1), lambda bh, r: (r, bh, 0, 0))],
        out_specs=pl.BlockSpec((1, 1, S, 1), lambda bh, r: (r, bh, 0, 0)),
        out_shape=jax.ShapeDtypeStruct((NH, BH, S, 1), jnp.int32),
    )(bucket4)


_NW = 32
_IPW = T // _NW
_CH = 512
_NCHK = S // _CH
_KJ = _CH // 128

@functools.cache
def _sc_kernels():
    mesh = plsc.VectorSubcoreMesh(core_axis_name="c", subcore_axis_name="s")

    @functools.partial(
        pl.kernel, mesh=mesh,
        out_type=jax.ShapeDtypeStruct((T * S, QKV), jnp.float32),
        scratch_types=[
            pltpu.VMEM((_KJ, 128), jnp.int32),
            pltpu.VMEM((_CH, QKV), jnp.float32),
            pltpu.SemaphoreType.DMA,
        ],
    )
    def sc_scatter(qkv_hbm, dest_hbm, sorted_hbm, idx_v, rows_v, sem):
        wid = lax.axis_index("s") * 2 + lax.axis_index("c")

        def inst_body(k, carry):
            t = wid * _IPW + k
            bh = lax.rem(t, BH)

            def chunk_body(c, carry2):
                pltpu.sync_copy(dest_hbm.at[t, pl.ds(c * _KJ, _KJ)], idx_v)
                pltpu.sync_copy(qkv_hbm.at[pl.ds(bh * S + c * _CH, _CH)], rows_v)
                cps = [
                    pltpu.async_copy(rows_v.at[pl.ds(j * 128, 128)],
                                     sorted_hbm.at[idx_v.at[j]], sem)
                    for j in range(_KJ)
                ]
                for cp in cps:
                    cp.wait()
                return carry2

            return lax.fori_loop(0, _NCHK, chunk_body, carry)

        lax.fori_loop(0, _IPW, inst_body, 0)

    @functools.partial(
        pl.kernel, mesh=mesh,
        out_type=jax.ShapeDtypeStruct((T * S, OLSE), jnp.float32),
        scratch_types=[
            pltpu.VMEM((_KJ, 128), jnp.int32),
            pltpu.VMEM((_CH, OLSE), jnp.float32),
            pltpu.SemaphoreType.DMA,
        ],
    )
    def sc_gather(olse_hbm, dest_hbm, out_hbm, idx_v, rows_v, sem):
        wid = lax.axis_index("s") * 2 + lax.axis_index("c")

        def inst_body(k, carry):
            t = wid * _IPW + k

            def chunk_body(c, carry2):
                pltpu.sync_copy(dest_hbm.at[t, pl.ds(c * _KJ, _KJ)], idx_v)
                cps = [
                    pltpu.async_copy(olse_hbm.at[idx_v.at[j]],
                                     rows_v.at[pl.ds(j * 128, 128)], sem)
                    for j in range(_KJ)
                ]
                for cp in cps:
                    cp.wait()
                pltpu.sync_copy(rows_v, out_hbm.at[pl.ds(t * S + c * _CH, _CH)])
                return carry2

            return lax.fori_loop(0, _NCHK, chunk_body, carry)

        lax.fori_loop(0, _IPW, inst_body, 0)

    return sc_scatter, sc_gather


_CPG = 8


def _attn_body(cur_ref, prev_ref, out_ref):
    scale = 1.0 / np.sqrt(DH)
    riota = lax.broadcasted_iota(jnp.int32, (BUCKET, 2 * BUCKET), 0)
    ciota = lax.broadcasted_iota(jnp.int32, (BUCKET, 2 * BUCKET), 1)
    self_mask = riota == ciota
    for i in range(_CPG):
        qkv_c = cur_ref[0, 0, i * BUCKET:(i + 1) * BUCKET, :]
        if i > 0:
            qkv_p = cur_ref[0, 0, (i - 1) * BUCKET:i * BUCKET, :]
        else:
            qkv_p = prev_ref[0, 0, (_CPG - 1) * BUCKET:_CPG * BUCKET, :]
        qc = qkv_c[:, :DH]
        vc = qkv_c[:, DH:]
        qp = qkv_p[:, :DH]
        vp = qkv_p[:, DH:]
        kc = qc / (jnp.sqrt(jnp.sum(qc * qc, axis=1, keepdims=True)) + 1e-6)
        kp = qp / (jnp.sqrt(jnp.sum(qp * qp, axis=1, keepdims=True)) + 1e-6)
        kcat = jnp.concatenate([kc, kp], axis=0)
        vcat = jnp.concatenate([vc, vp], axis=0)
        dots = lax.dot_general(qc, kcat, (((1,), (1,)), ((), ())),
                               preferred_element_type=jnp.float32) * scale
        dots = jnp.where(self_mask, -1e5, dots)
        m = jnp.max(dots, axis=1, keepdims=True)
        p = jnp.exp(dots - m)
        s = jnp.sum(p, axis=1, keepdims=True)
        lse = m + jnp.log(s)
        oc = lax.dot_general(p, vcat, (((1,), (0,)), ((), ())),
                             preferred_element_type=jnp.float32) / s
        out_ref[0, 0, i * BUCKET:(i + 1) * BUCKET, :] = jnp.concatenate(
            [oc, lse, jnp.zeros((BUCKET, OLSE - DH - 1), jnp.float32)], axis=1)


def _attn(srt, t_count):
    g2 = NCH // _CPG
    rows = _CPG * BUCKET
    return pl.pallas_call(
        _attn_body,
        grid=(t_count, g2),
        in_specs=[
            pl.BlockSpec((1, 1, rows, QKV), lambda t, j: (t, j, 0, 0)),
            pl.BlockSpec((1, 1, rows, QKV), lambda t, j: (t, (j + g2 - 1) % g2, 0, 0)),
        ],
        out_specs=pl.BlockSpec((1, 1, rows, OLSE), lambda t, j: (t, j, 0, 0)),
        out_shape=jax.ShapeDtypeStruct((t_count, g2, rows, OLSE), jnp.float32),
    )(srt, srt)


def _ffn_body(a_ref, wo_ref, w1_ref, b1_ref, w2_ref, b2_ref, o_ref):
    a = a_ref[...]
    x = jnp.dot(a, wo_ref[...], preferred_element_type=jnp.float32)
    h = jnp.maximum(
        jnp.dot(x, w1_ref[...], preferred_element_type=jnp.float32) + b1_ref[...],
        0.0,
    )
    o_ref[...] = jnp.dot(h, w2_ref[...], preferred_element_type=jnp.float32) + b2_ref[...]


def _ffn(attn, Wo, W1, b1, W2, b2):
    N = B * S
    TILE = 512
    out = pl.pallas_call(
        _ffn_body,
        grid=(N // TILE,),
        in_specs=[
            pl.BlockSpec((TILE, D), lambda i: (i, 0)),
            pl.BlockSpec((D, D), lambda i: (0, 0)),
            pl.BlockSpec((D, DFF), lambda i: (0, 0)),
            pl.BlockSpec((1, DFF), lambda i: (0, 0)),
            pl.BlockSpec((DFF, D), lambda i: (0, 0)),
            pl.BlockSpec((1, D), lambda i: (0, 0)),
        ],
        out_specs=pl.BlockSpec((TILE, D), lambda i: (i, 0)),
        out_shape=jax.ShapeDtypeStruct((N, D), jnp.float32),
    )(attn.reshape(N, D), Wo, W1, b1.reshape(1, DFF), W2, b2.reshape(1, D))
    return out


def kernel(src, Wqk, Wv, Wo, rot, W1, b1, W2, b2):
    qk = (src @ Wqk).reshape(B, S, H, DH).transpose(0, 2, 1, 3)
    v = (src @ Wv).reshape(B, S, H, DH).transpose(0, 2, 1, 3)
    qn = qk / (jnp.linalg.norm(qk, axis=-1, keepdims=True) + 1e-6)
    rotated = jnp.einsum('bhsd,dnr->bhsnr', qn, rot)
    buckets = jnp.argmax(
        jnp.concatenate([rotated, -rotated], axis=-1), axis=-1
    ).astype(jnp.int32)

    bucket4 = buckets.transpose(3, 0, 1, 2).reshape(NH, BH, S, 1)
    dest_g = _dest(bucket4)
    dest_sc = dest_g.reshape(T, S // 128, 128)

    sc_scatter, sc_gather = _sc_kernels()
    qkv = jnp.concatenate([qk, v], axis=-1).reshape(BH * S, QKV)
    srt = sc_scatter(qkv, dest_sc)

    olse = _attn(srt.reshape(T, NCH // _CPG, _CPG * BUCKET, QKV), T)
    uns = sc_gather(olse.reshape(T * S, OLSE), dest_sc).reshape(NH, B, H, S, OLSE)

    o_u = uns[..., :DH]
    l_u = uns[..., DH]
    w = jax.nn.softmax(l_u, axis=0)[..., None]
    attn = jnp.sum(o_u * w, axis=0)
    attn = attn.transpose(0, 2, 1, 3).reshape(B * S, D)
    return _ffn(attn, Wo, W1, b1, W2, b2).reshape(B, S, D)


# speedup vs baseline: 4.5310x; 1.3848x over previous
import functools

import jax
import jax.numpy as jnp
import numpy as np
from jax import lax
from jax.experimental import pallas as pl
from jax.experimental.pallas import tpu as pltpu
from jax.experimental.pallas import tpu_sc as plsc

B, S, D, H = 4, 4096, 1024, 16
DH = D // H
BUCKET = 64
NH = 4
NCH = S // BUCKET
NB = NCH
DFF = 4096
BH = B * H
T = NH * BH
SB = 512
QKV = 2 * DH
OLSE = 128


def _dest_body(bucket_ref, dest_ref):
    bh = pl.program_id(0)
    r = pl.program_id(1)
    bcol = bucket_ref[0, 0]
    lane = lax.broadcasted_iota(jnp.int32, (SB, BUCKET), 1)
    tri_s = (lax.broadcasted_iota(jnp.int32, (SB, SB), 0)
             > lax.broadcasted_iota(jnp.int32, (SB, SB), 1)).astype(jnp.bfloat16)
    tri_b = (lax.broadcasted_iota(jnp.int32, (BUCKET, BUCKET), 0)
             < lax.broadcasted_iota(jnp.int32, (BUCKET, BUCKET), 1)).astype(jnp.float32)
    tot = jnp.zeros((1, BUCKET), jnp.float32)
    ohs = []
    for j in range(S // SB):
        oh = (bcol[j * SB:(j + 1) * SB] == lane).astype(jnp.float32)
        ohs.append(oh)
        tot = tot + jnp.sum(oh, axis=0, keepdims=True)
    offs = jnp.dot(tot, tri_b, preferred_element_type=jnp.float32)
    tbase = ((r * BH + bh) * S).astype(jnp.float32)
    carry = jnp.zeros((1, BUCKET), jnp.float32)
    for j in range(S // SB):
        oh = ohs[j]
        cums = jnp.dot(tri_s, oh.astype(jnp.bfloat16),
                       preferred_element_type=jnp.float32)
        destc = jnp.sum(oh * (offs + carry + cums), axis=1, keepdims=True)
        dest_ref[0, 0, j * SB:(j + 1) * SB] = (destc + tbase).astype(jnp.int32)
        carry = carry + jnp.sum(oh, axis=0, keepdims=True)


def _dest(bucket4):
    return pl.pallas_call(
        _dest_body,
        grid=(BH, NH),
        in_specs=[pl.BlockSpec((1, 1, S, 1), lambda bh, r: (r, bh, 0, 0))],
        out_specs=pl.BlockSpec((1, 1, S, 1), lambda bh, r: (r, bh, 0, 0)),
        out_shape=jax.ShapeDtypeStruct((NH, BH, S, 1), jnp.int32),
    )(bucket4)


_NW = 32
_IPW = T // _NW
_CH = 512
_NCHK = S // _CH
_KJ = _CH // 128

@functools.cache
def _sc_kernels():
    mesh = plsc.VectorSubcoreMesh(core_axis_name="c", subcore_axis_name="s")

    @functools.partial(
        pl.kernel, mesh=mesh,
        out_type=jax.ShapeDtypeStruct((T * S, QKV), jnp.float32),
        scratch_types=[
            pltpu.VMEM((_KJ, 128), jnp.int32),
            pltpu.VMEM((_CH, QKV), jnp.float32),
            pltpu.SemaphoreType.DMA,
        ],
    )
    def sc_scatter(qkv_hbm, dest_hbm, sorted_hbm, idx_v, rows_v, sem):
        wid = lax.axis_index("s") * 2 + lax.axis_index("c")

        def inst_body(k, carry):
            t = wid * _IPW + k
            bh = lax.rem(t, BH)

            def chunk_body(c, carry2):
                pltpu.sync_copy(dest_hbm.at[t, pl.ds(c * _KJ, _KJ)], idx_v)
                pltpu.sync_copy(qkv_hbm.at[pl.ds(bh * S + c * _CH, _CH)], rows_v)
                cps = [
                    pltpu.async_copy(rows_v.at[pl.ds(j * 128, 128)],
                                     sorted_hbm.at[idx_v.at[j]], sem)
                    for j in range(_KJ)
                ]
                for cp in cps:
                    cp.wait()
                return carry2

            return lax.fori_loop(0, _NCHK, chunk_body, carry)

        lax.fori_loop(0, _IPW, inst_body, 0)

    @functools.partial(
        pl.kernel, mesh=mesh,
        out_type=jax.ShapeDtypeStruct((T * S, OLSE), jnp.float32),
        scratch_types=[
            pltpu.VMEM((_KJ, 128), jnp.int32),
            pltpu.VMEM((_CH, OLSE), jnp.float32),
            pltpu.SemaphoreType.DMA,
        ],
    )
    def sc_gather(olse_hbm, dest_hbm, out_hbm, idx_v, rows_v, sem):
        wid = lax.axis_index("s") * 2 + lax.axis_index("c")

        def inst_body(k, carry):
            t = wid * _IPW + k

            def chunk_body(c, carry2):
                pltpu.sync_copy(dest_hbm.at[t, pl.ds(c * _KJ, _KJ)], idx_v)
                cps = [
                    pltpu.async_copy(olse_hbm.at[idx_v.at[j]],
                                     rows_v.at[pl.ds(j * 128, 128)], sem)
                    for j in range(_KJ)
                ]
                for cp in cps:
                    cp.wait()
                pltpu.sync_copy(rows_v, out_hbm.at[pl.ds(t * S + c * _CH, _CH)])
                return carry2

            return lax.fori_loop(0, _NCHK, chunk_body, carry)

        lax.fori_loop(0, _IPW, inst_body, 0)

    return sc_scatter, sc_gather


_CPG = 8


_ROWS = _CPG * BUCKET
_KW = _ROWS + BUCKET


def _attn_body(cur_ref, prev_ref, out_ref):
    scale = 1.0 / np.sqrt(DH)
    cur = cur_ref[0, 0]
    halo = prev_ref[0, 0, (_CPG - 1) * BUCKET:, :]
    kwin = jnp.concatenate([halo, cur], axis=0)
    kq = kwin[:, :DH]
    kc = kq / (jnp.sqrt(jnp.sum(kq * kq, axis=1, keepdims=True)) + 1e-6)
    vwin = kwin[:, DH:]
    q = cur[:, :DH]
    dots = lax.dot_general(q, kc, (((1,), (1,)), ((), ())),
                           preferred_element_type=jnp.float32) * scale
    ri = lax.broadcasted_iota(jnp.int32, (_ROWS, _KW), 0)
    ci = lax.broadcasted_iota(jnp.int32, (_ROWS, _KW), 1)
    qch = ri // BUCKET
    kch = ci // BUCKET
    masked = (ci == ri + BUCKET) | ((kch != qch) & (kch != qch + 1))
    dots = jnp.where(masked, -1e5, dots)
    m = jnp.max(dots, axis=1, keepdims=True)
    p = jnp.exp(dots - m)
    s = jnp.sum(p, axis=1, keepdims=True)
    lse = m + jnp.log(s)
    oc = lax.dot_general(p, vwin, (((1,), (0,)), ((), ())),
                         preferred_element_type=jnp.float32) / s
    out_ref[0, 0] = jnp.concatenate(
        [oc, lse, jnp.zeros((_ROWS, OLSE - DH - 1), jnp.float32)], axis=1)


def _attn(srt, t_count):
    g2 = NCH // _CPG
    rows = _CPG * BUCKET
    return pl.pallas_call(
        _attn_body,
        grid=(t_count, g2),
        in_specs=[
            pl.BlockSpec((1, 1, rows, QKV), lambda t, j: (t, j, 0, 0)),
            pl.BlockSpec((1, 1, rows, QKV), lambda t, j: (t, (j + g2 - 1) % g2, 0, 0)),
        ],
        out_specs=pl.BlockSpec((1, 1, rows, OLSE), lambda t, j: (t, j, 0, 0)),
        out_shape=jax.ShapeDtypeStruct((t_count, g2, rows, OLSE), jnp.float32),
    )(srt, srt)


def _ffn_body(a_ref, wo_ref, w1_ref, b1_ref, w2_ref, b2_ref, o_ref):
    a = a_ref[...]
    x = jnp.dot(a, wo_ref[...], preferred_element_type=jnp.float32)
    h = jnp.maximum(
        jnp.dot(x, w1_ref[...], preferred_element_type=jnp.float32) + b1_ref[...],
        0.0,
    )
    o_ref[...] = jnp.dot(h, w2_ref[...], preferred_element_type=jnp.float32) + b2_ref[...]


def _ffn(attn, Wo, W1, b1, W2, b2):
    N = B * S
    TILE = 512
    out = pl.pallas_call(
        _ffn_body,
        grid=(N // TILE,),
        in_specs=[
            pl.BlockSpec((TILE, D), lambda i: (i, 0)),
            pl.BlockSpec((D, D), lambda i: (0, 0)),
            pl.BlockSpec((D, DFF), lambda i: (0, 0)),
            pl.BlockSpec((1, DFF), lambda i: (0, 0)),
            pl.BlockSpec((DFF, D), lambda i: (0, 0)),
            pl.BlockSpec((1, D), lambda i: (0, 0)),
        ],
        out_specs=pl.BlockSpec((TILE, D), lambda i: (i, 0)),
        out_shape=jax.ShapeDtypeStruct((N, D), jnp.float32),
    )(attn.reshape(N, D), Wo, W1, b1.reshape(1, DFF), W2, b2.reshape(1, D))
    return out


def kernel(src, Wqk, Wv, Wo, rot, W1, b1, W2, b2):
    qk = (src @ Wqk).reshape(B, S, H, DH).transpose(0, 2, 1, 3)
    v = (src @ Wv).reshape(B, S, H, DH).transpose(0, 2, 1, 3)
    qn = qk / (jnp.linalg.norm(qk, axis=-1, keepdims=True) + 1e-6)
    rotated = jnp.einsum('bhsd,dnr->bhsnr', qn, rot)
    buckets = jnp.argmax(
        jnp.concatenate([rotated, -rotated], axis=-1), axis=-1
    ).astype(jnp.int32)

    bucket4 = buckets.transpose(3, 0, 1, 2).reshape(NH, BH, S, 1)
    dest_g = _dest(bucket4)
    dest_sc = dest_g.reshape(T, S // 128, 128)

    sc_scatter, sc_gather = _sc_kernels()
    qkv = jnp.concatenate([qk, v], axis=-1).reshape(BH * S, QKV)
    srt = sc_scatter(qkv, dest_sc)

    olse = _attn(srt.reshape(T, NCH // _CPG, _CPG * BUCKET, QKV), T)
    uns = sc_gather(olse.reshape(T * S, OLSE), dest_sc).reshape(NH, B, H, S, OLSE)

    o_u = uns[..., :DH]
    l_u = uns[..., DH]
    w = jax.nn.softmax(l_u, axis=0)[..., None]
    attn = jnp.sum(o_u * w, axis=0)
    attn = attn.transpose(0, 2, 1, 3).reshape(B * S, D)
    return _ffn(attn, Wo, W1, b1, W2, b2).reshape(B, S, D)
